# baseline (device time: 58137 ns/iter reference)
import jax
import jax.numpy as jnp
from jax import lax
from jax.experimental import pallas as pl
from jax.experimental.pallas import tpu as pltpu

N_DEV = 32
SQ = 512
D = 1024
NH = 8
DH = 128
BLK = 4
NBLK = N_DEV // BLK
R1 = SQ // BLK
R2 = R1 // NBLK
SCALE = 0.08838834764831843


def kernel(x, Wq, Wo, Wk, Wv):
    def body(x_ref, wq_ref, wk_ref, wv_ref, wo_ref, out_ref,
             qbuf, pbuf, l1_buf, bbuf, l2_buf, ag_buf,
             l1rs_sems, l2rs_sems, l2ag_sems, l1ag_sems,
             s_l1rs, s_l2rs, s_l2ag, s_l1ag):
        my = lax.axis_index("i")
        b = my // BLK
        r = my % BLK

        xb = x_ref[0].astype(jnp.bfloat16)
        q = jnp.dot(xb, wq_ref[...].astype(jnp.bfloat16),
                    preferred_element_type=jnp.float32).astype(jnp.bfloat16)
        qbuf[...] = q
        k = jnp.dot(xb, wk_ref[...].astype(jnp.bfloat16),
                    preferred_element_type=jnp.float32).astype(jnp.bfloat16)
        v = jnp.dot(xb, wv_ref[...].astype(jnp.bfloat16),
                    preferred_element_type=jnp.float32).astype(jnp.bfloat16)
        wo_b = wo_ref[...].astype(jnp.bfloat16)

        for t in range(BLK):
            g = lax.rem(r + 1 + t, BLK)
            r0 = g * R1
            qg = qbuf[pl.ds(r0, R1), :]
            outs = []
            for h in range(NH):
                qh = qg[:, h * DH:(h + 1) * DH]
                kh = k[:, h * DH:(h + 1) * DH]
                vh = v[:, h * DH:(h + 1) * DH]
                s = lax.dot_general(qh, kh, (((1,), (1,)), ((), ())),
                                    preferred_element_type=jnp.float32) * SCALE
                m = jnp.max(s, axis=-1, keepdims=True)
                p = jnp.exp(s - m)
                l = jnp.sum(p, axis=-1, keepdims=True)
                oh = lax.dot_general(p.astype(jnp.bfloat16), vh,
                                     (((1,), (0,)), ((), ())),
                                     preferred_element_type=jnp.float32)
                outs.append(oh / l)
            attn_g = jnp.concatenate(outs, axis=1).astype(jnp.bfloat16)
            partial_g = jnp.dot(attn_g, wo_b,
                                preferred_element_type=jnp.float32)
            if t < BLK - 1:
                pbuf[pl.ds(r0, R1), :] = partial_g.astype(jnp.bfloat16)
                rdma = pltpu.make_async_remote_copy(
                    src_ref=pbuf.at[pl.ds(r0, R1), :],
                    dst_ref=l1_buf.at[r],
                    send_sem=s_l1rs.at[g],
                    recv_sem=l1rs_sems.at[r],
                    device_id=(b * BLK + g,),
                    device_id_type=pl.DeviceIdType.MESH,
                )
                rdma.start()
            else:
                l1_buf[r] = partial_g.astype(jnp.bfloat16)

        for t in range(1, BLK):
            rr = lax.rem(r + t, BLK)
            recv = pltpu.make_async_remote_copy(
                src_ref=pbuf.at[pl.ds(0, R1), :],
                dst_ref=l1_buf.at[rr],
                send_sem=l1rs_sems.at[rr],
                recv_sem=l1rs_sems.at[rr],
                device_id=(my,),
                device_id_type=pl.DeviceIdType.MESH,
            )
            recv.wait_recv()
        bsum = jnp.sum(l1_buf[...].astype(jnp.float32), axis=0)
        bbuf[...] = bsum.astype(jnp.bfloat16)

        for t in range(1, NBLK):
            c = lax.rem(b + t, NBLK)
            rdma = pltpu.make_async_remote_copy(
                src_ref=bbuf.at[pl.ds(c * R2, R2), :],
                dst_ref=l2_buf.at[b],
                send_sem=s_l2rs.at[c],
                recv_sem=l2rs_sems.at[b],
                device_id=(c * BLK + r,),
                device_id_type=pl.DeviceIdType.MESH,
            )
            rdma.start()
        l2_buf[b] = bbuf[pl.ds(b * R2, R2), :]
        for t in range(1, NBLK):
            c = lax.rem(b + t, NBLK)
            recv = pltpu.make_async_remote_copy(
                src_ref=bbuf.at[pl.ds(0, R2), :],
                dst_ref=l2_buf.at[c],
                send_sem=l2rs_sems.at[c],
                recv_sem=l2rs_sems.at[c],
                device_id=(my,),
                device_id_type=pl.DeviceIdType.MESH,
            )
            recv.wait_recv()
        final16 = jnp.sum(l2_buf[...].astype(jnp.float32), axis=0)
        ag_buf[pl.ds(r * R1 + b * R2, R2), :] = final16.astype(jnp.bfloat16)

        for t in range(1, NBLK):
            c = lax.rem(b + t, NBLK)
            rdma = pltpu.make_async_remote_copy(
                src_ref=ag_buf.at[pl.ds(r * R1 + b * R2, R2), :],
                dst_ref=ag_buf.at[pl.ds(r * R1 + b * R2, R2), :],
                send_sem=s_l2ag.at[c],
                recv_sem=l2ag_sems.at[b],
                device_id=(c * BLK + r,),
                device_id_type=pl.DeviceIdType.MESH,
            )
            rdma.start()
        for t in range(1, NBLK):
            c = lax.rem(b + t, NBLK)
            recv = pltpu.make_async_remote_copy(
                src_ref=ag_buf.at[pl.ds(0, R2), :],
                dst_ref=ag_buf.at[pl.ds(r * R1 + c * R2, R2), :],
                send_sem=l2ag_sems.at[c],
                recv_sem=l2ag_sems.at[c],
                device_id=(my,),
                device_id_type=pl.DeviceIdType.MESH,
            )
            recv.wait_recv()

        for t in range(1, BLK):
            rr = lax.rem(r + t, BLK)
            rdma = pltpu.make_async_remote_copy(
                src_ref=ag_buf.at[pl.ds(r * R1, R1), :],
                dst_ref=ag_buf.at[pl.ds(r * R1, R1), :],
                send_sem=s_l1ag.at[rr],
                recv_sem=l1ag_sems.at[r],
                device_id=(b * BLK + rr,),
                device_id_type=pl.DeviceIdType.MESH,
            )
            rdma.start()
        for t in range(1, BLK):
            rr = lax.rem(r + t, BLK)
            recv = pltpu.make_async_remote_copy(
                src_ref=ag_buf.at[pl.ds(0, R1), :],
                dst_ref=ag_buf.at[pl.ds(rr * R1, R1), :],
                send_sem=l1ag_sems.at[rr],
                recv_sem=l1ag_sems.at[rr],
                device_id=(my,),
                device_id_type=pl.DeviceIdType.MESH,
            )
            recv.wait_recv()

        for t in range(1, BLK):
            g = lax.rem(r + t, BLK)
            snd = pltpu.make_async_remote_copy(
                src_ref=pbuf.at[pl.ds(0, R1), :],
                dst_ref=l1_buf.at[r],
                send_sem=s_l1rs.at[g],
                recv_sem=l1rs_sems.at[r],
                device_id=(my,),
                device_id_type=pl.DeviceIdType.MESH,
            )
            snd.wait_send()
            snd2 = pltpu.make_async_remote_copy(
                src_ref=ag_buf.at[pl.ds(0, R1), :],
                dst_ref=ag_buf.at[pl.ds(0, R1), :],
                send_sem=s_l1ag.at[g],
                recv_sem=l1ag_sems.at[r],
                device_id=(my,),
                device_id_type=pl.DeviceIdType.MESH,
            )
            snd2.wait_send()
        for t in range(1, NBLK):
            c = lax.rem(b + t, NBLK)
            snd = pltpu.make_async_remote_copy(
                src_ref=bbuf.at[pl.ds(0, R2), :],
                dst_ref=l2_buf.at[b],
                send_sem=s_l2rs.at[c],
                recv_sem=l2rs_sems.at[b],
                device_id=(my,),
                device_id_type=pl.DeviceIdType.MESH,
            )
            snd.wait_send()
            snd2 = pltpu.make_async_remote_copy(
                src_ref=ag_buf.at[pl.ds(0, R2), :],
                dst_ref=ag_buf.at[pl.ds(0, R2), :],
                send_sem=s_l2ag.at[c],
                recv_sem=l2ag_sems.at[b],
                device_id=(my,),
                device_id_type=pl.DeviceIdType.MESH,
            )
            snd2.wait_send()

        out_ref[0] = ag_buf[...].astype(jnp.float32)

    return pl.pallas_call(
        body,
        out_shape=jax.ShapeDtypeStruct((1, SQ, D), jnp.float32),
        in_specs=[pl.BlockSpec(memory_space=pltpu.VMEM)] * 5,
        out_specs=pl.BlockSpec(memory_space=pltpu.VMEM),
        scratch_shapes=[
            pltpu.VMEM((SQ, D), jnp.bfloat16),
            pltpu.VMEM((SQ, D), jnp.bfloat16),
            pltpu.VMEM((BLK, R1, D), jnp.bfloat16),
            pltpu.VMEM((R1, D), jnp.bfloat16),
            pltpu.VMEM((NBLK, R2, D), jnp.bfloat16),
            pltpu.VMEM((SQ, D), jnp.bfloat16),
            pltpu.SemaphoreType.DMA((BLK,)),
            pltpu.SemaphoreType.DMA((NBLK,)),
            pltpu.SemaphoreType.DMA((NBLK,)),
            pltpu.SemaphoreType.DMA((BLK,)),
            pltpu.SemaphoreType.DMA((BLK,)),
            pltpu.SemaphoreType.DMA((NBLK,)),
            pltpu.SemaphoreType.DMA((NBLK,)),
            pltpu.SemaphoreType.DMA((BLK,)),
        ],
    )(x, Wq, Wk, Wv, Wo)


# device time: 55159 ns/iter; 1.0540x vs baseline; 1.0540x over previous
import jax
import jax.numpy as jnp
from jax import lax
from jax.experimental import pallas as pl
from jax.experimental.pallas import tpu as pltpu

N_DEV = 32
SQ = 512
D = 1024
NH = 8
DH = 128
BLK = 4
NBLK = N_DEV // BLK
R1 = SQ // BLK
R2 = R1 // NBLK
SCALE = 0.08838834764831843


def kernel(x, Wq, Wo, Wk, Wv):
    def body(x_hbm, wq_hbm, wk_hbm, wv_hbm, wo_hbm, out_ref,
             xv, wqv, wkv, wvv, wov,
             qbuf, pbuf, l1_buf, bbuf, l2_buf, ag_buf,
             in_sems,
             l1rs_sems, l2rs_sems, l2ag_sems, l1ag_sems,
             s_l1rs, s_l2rs, s_l2ag, s_l1ag):
        my = lax.axis_index("i")
        b = my // BLK
        r = my % BLK

        ins = [(x_hbm, xv), (wq_hbm, wqv), (wk_hbm, wkv),
               (wv_hbm, wvv), (wo_hbm, wov)]
        for i, (src, dst) in enumerate(ins):
            pltpu.make_async_copy(src, dst, in_sems.at[i]).start()

        barrier_sem = pltpu.get_barrier_semaphore()
        for t in range(1, BLK):
            pl.semaphore_signal(
                barrier_sem, inc=1,
                device_id=(b * BLK + lax.rem(r + t, BLK),),
                device_id_type=pl.DeviceIdType.MESH,
            )
        for t in range(1, NBLK):
            pl.semaphore_signal(
                barrier_sem, inc=1,
                device_id=(lax.rem(b + t, NBLK) * BLK + r,),
                device_id_type=pl.DeviceIdType.MESH,
            )
        pl.semaphore_wait(barrier_sem, (BLK - 1) + (NBLK - 1))

        pltpu.make_async_copy(x_hbm, xv, in_sems.at[0]).wait()
        xb = xv[0].astype(jnp.bfloat16)
        pltpu.make_async_copy(wq_hbm, wqv, in_sems.at[1]).wait()
        q = jnp.dot(xb, wqv[...].astype(jnp.bfloat16),
                    preferred_element_type=jnp.float32).astype(jnp.bfloat16)
        qbuf[...] = q
        pltpu.make_async_copy(wk_hbm, wkv, in_sems.at[2]).wait()
        k = jnp.dot(xb, wkv[...].astype(jnp.bfloat16),
                    preferred_element_type=jnp.float32).astype(jnp.bfloat16)
        pltpu.make_async_copy(wv_hbm, wvv, in_sems.at[3]).wait()
        v = jnp.dot(xb, wvv[...].astype(jnp.bfloat16),
                    preferred_element_type=jnp.float32).astype(jnp.bfloat16)
        pltpu.make_async_copy(wo_hbm, wov, in_sems.at[4]).wait()
        wo_b = wov[...].astype(jnp.bfloat16)

        for t in range(BLK):
            g = lax.rem(r + 1 + t, BLK)
            r0 = g * R1
            qg = qbuf[pl.ds(r0, R1), :]
            outs = []
            for h in range(NH):
                qh = qg[:, h * DH:(h + 1) * DH]
                kh = k[:, h * DH:(h + 1) * DH]
                vh = v[:, h * DH:(h + 1) * DH]
                s = lax.dot_general(qh, kh, (((1,), (1,)), ((), ())),
                                    preferred_element_type=jnp.float32) * SCALE
                m = jnp.max(s, axis=-1, keepdims=True)
                p = jnp.exp(s - m)
                l = jnp.sum(p, axis=-1, keepdims=True)
                oh = lax.dot_general(p.astype(jnp.bfloat16), vh,
                                     (((1,), (0,)), ((), ())),
                                     preferred_element_type=jnp.float32)
                outs.append(oh / l)
            attn_g = jnp.concatenate(outs, axis=1).astype(jnp.bfloat16)
            partial_g = jnp.dot(attn_g, wo_b,
                                preferred_element_type=jnp.float32)
            if t < BLK - 1:
                pbuf[pl.ds(r0, R1), :] = partial_g.astype(jnp.bfloat16)
                rdma = pltpu.make_async_remote_copy(
                    src_ref=pbuf.at[pl.ds(r0, R1), :],
                    dst_ref=l1_buf.at[r],
                    send_sem=s_l1rs.at[g],
                    recv_sem=l1rs_sems.at[r],
                    device_id=(b * BLK + g,),
                    device_id_type=pl.DeviceIdType.MESH,
                )
                rdma.start()
            else:
                l1_buf[r] = partial_g.astype(jnp.bfloat16)

        for t in range(1, BLK):
            rr = lax.rem(r + t, BLK)
            recv = pltpu.make_async_remote_copy(
                src_ref=pbuf.at[pl.ds(0, R1), :],
                dst_ref=l1_buf.at[rr],
                send_sem=l1rs_sems.at[rr],
                recv_sem=l1rs_sems.at[rr],
                device_id=(my,),
                device_id_type=pl.DeviceIdType.MESH,
            )
            recv.wait_recv()
        bsum = jnp.sum(l1_buf[...].astype(jnp.float32), axis=0)
        bbuf[...] = bsum.astype(jnp.bfloat16)

        for t in range(1, NBLK):
            c = lax.rem(b + t, NBLK)
            rdma = pltpu.make_async_remote_copy(
                src_ref=bbuf.at[pl.ds(c * R2, R2), :],
                dst_ref=l2_buf.at[b],
                send_sem=s_l2rs.at[c],
                recv_sem=l2rs_sems.at[b],
                device_id=(c * BLK + r,),
                device_id_type=pl.DeviceIdType.MESH,
            )
            rdma.start()
        l2_buf[b] = bbuf[pl.ds(b * R2, R2), :]
        for t in range(1, NBLK):
            c = lax.rem(b + t, NBLK)
            recv = pltpu.make_async_remote_copy(
                src_ref=bbuf.at[pl.ds(0, R2), :],
                dst_ref=l2_buf.at[c],
                send_sem=l2rs_sems.at[c],
                recv_sem=l2rs_sems.at[c],
                device_id=(my,),
                device_id_type=pl.DeviceIdType.MESH,
            )
            recv.wait_recv()
        final16 = jnp.sum(l2_buf[...].astype(jnp.float32), axis=0)
        ag_buf[pl.ds(r * R1 + b * R2, R2), :] = final16.astype(jnp.bfloat16)

        for t in range(1, NBLK):
            c = lax.rem(b + t, NBLK)
            rdma = pltpu.make_async_remote_copy(
                src_ref=ag_buf.at[pl.ds(r * R1 + b * R2, R2), :],
                dst_ref=ag_buf.at[pl.ds(r * R1 + b * R2, R2), :],
                send_sem=s_l2ag.at[c],
                recv_sem=l2ag_sems.at[b],
                device_id=(c * BLK + r,),
                device_id_type=pl.DeviceIdType.MESH,
            )
            rdma.start()
        for t in range(1, NBLK):
            c = lax.rem(b + t, NBLK)
            recv = pltpu.make_async_remote_copy(
                src_ref=ag_buf.at[pl.ds(0, R2), :],
                dst_ref=ag_buf.at[pl.ds(r * R1 + c * R2, R2), :],
                send_sem=l2ag_sems.at[c],
                recv_sem=l2ag_sems.at[c],
                device_id=(my,),
                device_id_type=pl.DeviceIdType.MESH,
            )
            recv.wait_recv()

        for t in range(1, BLK):
            rr = lax.rem(r + t, BLK)
            rdma = pltpu.make_async_remote_copy(
                src_ref=ag_buf.at[pl.ds(r * R1, R1), :],
                dst_ref=ag_buf.at[pl.ds(r * R1, R1), :],
                send_sem=s_l1ag.at[rr],
                recv_sem=l1ag_sems.at[r],
                device_id=(b * BLK + rr,),
                device_id_type=pl.DeviceIdType.MESH,
            )
            rdma.start()
        for t in range(1, BLK):
            rr = lax.rem(r + t, BLK)
            recv = pltpu.make_async_remote_copy(
                src_ref=ag_buf.at[pl.ds(0, R1), :],
                dst_ref=ag_buf.at[pl.ds(rr * R1, R1), :],
                send_sem=l1ag_sems.at[rr],
                recv_sem=l1ag_sems.at[rr],
                device_id=(my,),
                device_id_type=pl.DeviceIdType.MESH,
            )
            recv.wait_recv()

        for t in range(1, BLK):
            g = lax.rem(r + t, BLK)
            snd = pltpu.make_async_remote_copy(
                src_ref=pbuf.at[pl.ds(0, R1), :],
                dst_ref=l1_buf.at[r],
                send_sem=s_l1rs.at[g],
                recv_sem=l1rs_sems.at[r],
                device_id=(my,),
                device_id_type=pl.DeviceIdType.MESH,
            )
            snd.wait_send()
            snd2 = pltpu.make_async_remote_copy(
                src_ref=ag_buf.at[pl.ds(0, R1), :],
                dst_ref=ag_buf.at[pl.ds(0, R1), :],
                send_sem=s_l1ag.at[g],
                recv_sem=l1ag_sems.at[r],
                device_id=(my,),
                device_id_type=pl.DeviceIdType.MESH,
            )
            snd2.wait_send()
        for t in range(1, NBLK):
            c = lax.rem(b + t, NBLK)
            snd = pltpu.make_async_remote_copy(
                src_ref=bbuf.at[pl.ds(0, R2), :],
                dst_ref=l2_buf.at[b],
                send_sem=s_l2rs.at[c],
                recv_sem=l2rs_sems.at[b],
                device_id=(my,),
                device_id_type=pl.DeviceIdType.MESH,
            )
            snd.wait_send()
            snd2 = pltpu.make_async_remote_copy(
                src_ref=ag_buf.at[pl.ds(0, R2), :],
                dst_ref=ag_buf.at[pl.ds(0, R2), :],
                send_sem=s_l2ag.at[c],
                recv_sem=l2ag_sems.at[b],
                device_id=(my,),
                device_id_type=pl.DeviceIdType.MESH,
            )
            snd2.wait_send()

        out_ref[0] = ag_buf[...].astype(jnp.float32)

    return pl.pallas_call(
        body,
        out_shape=jax.ShapeDtypeStruct((1, SQ, D), jnp.float32),
        in_specs=[pl.BlockSpec(memory_space=pltpu.MemorySpace.HBM)] * 5,
        out_specs=pl.BlockSpec(memory_space=pltpu.VMEM),
        scratch_shapes=[
            pltpu.VMEM((1, SQ, D), jnp.float32),
            pltpu.VMEM((D, D), jnp.float32),
            pltpu.VMEM((D, D), jnp.float32),
            pltpu.VMEM((D, D), jnp.float32),
            pltpu.VMEM((D, D), jnp.float32),
            pltpu.VMEM((SQ, D), jnp.bfloat16),
            pltpu.VMEM((SQ, D), jnp.bfloat16),
            pltpu.VMEM((BLK, R1, D), jnp.bfloat16),
            pltpu.VMEM((R1, D), jnp.bfloat16),
            pltpu.VMEM((NBLK, R2, D), jnp.bfloat16),
            pltpu.VMEM((SQ, D), jnp.bfloat16),
            pltpu.SemaphoreType.DMA((5,)),
            pltpu.SemaphoreType.DMA((BLK,)),
            pltpu.SemaphoreType.DMA((NBLK,)),
            pltpu.SemaphoreType.DMA((NBLK,)),
            pltpu.SemaphoreType.DMA((BLK,)),
            pltpu.SemaphoreType.DMA((BLK,)),
            pltpu.SemaphoreType.DMA((NBLK,)),
            pltpu.SemaphoreType.DMA((NBLK,)),
            pltpu.SemaphoreType.DMA((BLK,)),
        ],
        compiler_params=pltpu.CompilerParams(collective_id=0),
    )(x, Wq, Wk, Wv, Wo)
